# baseline (speedup 1.0000x reference)
import jax
import jax.numpy as jnp
from jax import lax
from jax.experimental import pallas as pl
from jax.experimental.pallas import tpu as pltpu
from jax.experimental.pallas import tpu_sc as plsc

EMB = 32
NC, NS = 2, 16
NW = NC * NS
TC_BLK = 8192


def _retile_body(in_ref, out_ref):
    out_ref[...] = pltpu.einshape("e(ab)->a(be)", in_ref[...], b=4)


def _retile(W):
    V = W.shape[0]
    g = (V + TC_BLK - 1) // TC_BLK
    Wt = W.T
    flat = pl.pallas_call(
        _retile_body,
        grid=(g,),
        in_specs=[pl.BlockSpec((EMB, TC_BLK), lambda i: (0, i))],
        out_specs=pl.BlockSpec((TC_BLK // 4, 128), lambda i: (i, 0)),
        out_shape=jax.ShapeDtypeStruct((V * EMB // 128, 128), jnp.float32),
    )(Wt)
    return flat.reshape(V, EMB)


def _ctx_gather_body(ctx_hbm, win_hbm, ctx_out, slab_c, rows_v,
                     sem_i, sem_g, sem_o):
    B = ctx_hbm.shape[0] // 6
    n = B // NW
    wid = lax.axis_index("s") * NC + lax.axis_index("c")
    base = wid * n
    pltpu.async_copy(ctx_hbm.at[pl.ds(6 * base, 6 * n)], slab_c, sem_i).wait()
    out_copies = [None, None]
    for k in range(6):
        buf = k % 2
        if out_copies[buf] is not None:
            out_copies[buf].wait()
        idx = slab_c.at[pl.ds(k * n, n)]
        pltpu.async_copy(win_hbm.at[idx], rows_v.at[buf], sem_g).wait()
        out_copies[buf] = pltpu.async_copy(
            rows_v.at[buf], ctx_out.at[pl.ds(6 * base + k * n, n)], sem_o)
    out_copies[0].wait()
    out_copies[1].wait()


def _tn_gather_body(tgt_hbm, neg_hbm, wout_hbm, tgt_out, neg_out,
                    slab_t, slab_n, rows_v, sem_i, sem_g, sem_o):
    B = tgt_hbm.shape[0]
    n = B // NW
    wid = lax.axis_index("s") * NC + lax.axis_index("c")
    base = wid * n
    ti = pltpu.async_copy(tgt_hbm.at[pl.ds(base, n)], slab_t, sem_i)
    ni = pltpu.async_copy(neg_hbm.at[pl.ds(5 * base, 5 * n)], slab_n, sem_i)
    ti.wait()
    ni.wait()
    chunks = ([(slab_t, 0, tgt_out, base)]
              + [(slab_n, k, neg_out, 5 * base + k * n) for k in range(5)])
    out_copies = [None, None]
    for i, (slab, k, out, off) in enumerate(chunks):
        buf = i % 2
        if out_copies[buf] is not None:
            out_copies[buf].wait()
        idx = slab.at[pl.ds(k * n, n)]
        pltpu.async_copy(wout_hbm.at[idx], rows_v.at[buf], sem_g).wait()
        out_copies[buf] = pltpu.async_copy(
            rows_v.at[buf], out.at[pl.ds(off, n)], sem_o)
    out_copies[0].wait()
    out_copies[1].wait()


def _dense_body(ctx_ref, tgt_ref, neg_ref, out_ref):
    @pl.when(pl.program_id(0) == 0)
    def _():
        out_ref[0, 0] = jnp.float32(0.0)
        out_ref[0, 1] = jnp.float32(0.0)

    eps = 1e-12
    cm = jnp.sum(ctx_ref[...], axis=1) * (1.0 / 6.0)
    t = tgt_ref[...]
    tt = jnp.sum(t * t, axis=1)
    cc = jnp.sum(cm * cm, axis=1)
    tc = jnp.sum(t * cm, axis=1)
    rt = lax.rsqrt(jnp.maximum(tt, eps))
    rc = lax.rsqrt(jnp.maximum(cc, eps))
    pos = jnp.sum(jax.nn.sigmoid(tc * rt * rc))
    neg = jnp.float32(0.0)
    for j in range(5):
        nrow = neg_ref[:, j, :]
        nn = jnp.sum(nrow * nrow, axis=1)
        tn = jnp.sum(t * nrow, axis=1)
        rn = lax.rsqrt(jnp.maximum(nn, eps))
        neg = neg + jnp.sum(jax.nn.sigmoid(-(tn * rt * rn)))
    out_ref[0, 0] += pos
    out_ref[0, 1] += neg


def kernel(contexts, target, negatives, W_in, W_out):
    B = contexts.shape[0]
    n = B // NW
    mesh = plsc.VectorSubcoreMesh(core_axis_name="c", subcore_axis_name="s")

    win_rm = _retile(W_in)
    ctx_rows = pl.kernel(
        _ctx_gather_body,
        out_type=jax.ShapeDtypeStruct((6 * B, EMB), jnp.float32),
        mesh=mesh,
        scratch_types=[
            pltpu.VMEM((6 * n,), jnp.int32),
            pltpu.VMEM((2, n, EMB), jnp.float32),
            pltpu.SemaphoreType.DMA,
            pltpu.SemaphoreType.DMA,
            pltpu.SemaphoreType.DMA,
        ],
        compiler_params=pltpu.CompilerParams(use_tc_tiling_on_sc=False),
    )(contexts.reshape(-1).astype(jnp.int32), win_rm)

    wout_rm = _retile(W_out)
    tgt_rows, neg_rows = pl.kernel(
        _tn_gather_body,
        out_type=(
            jax.ShapeDtypeStruct((B, EMB), jnp.float32),
            jax.ShapeDtypeStruct((5 * B, EMB), jnp.float32),
        ),
        mesh=mesh,
        scratch_types=[
            pltpu.VMEM((n,), jnp.int32),
            pltpu.VMEM((5 * n,), jnp.int32),
            pltpu.VMEM((2, n, EMB), jnp.float32),
            pltpu.SemaphoreType.DMA,
            pltpu.SemaphoreType.DMA,
            pltpu.SemaphoreType.DMA,
        ],
        compiler_params=pltpu.CompilerParams(use_tc_tiling_on_sc=False),
    )(target.reshape(-1).astype(jnp.int32),
      negatives.reshape(-1).astype(jnp.int32), wout_rm)

    ctx3 = ctx_rows.reshape(B, 6, EMB)
    neg3 = neg_rows.reshape(B, 5, EMB)
    R = 2048
    partial = pl.pallas_call(
        _dense_body,
        grid=(B // R,),
        in_specs=[
            pl.BlockSpec((R, 6, EMB), lambda i: (i, 0, 0)),
            pl.BlockSpec((R, EMB), lambda i: (i, 0)),
            pl.BlockSpec((R, 5, EMB), lambda i: (i, 0, 0)),
        ],
        out_specs=pl.BlockSpec((1, 2), lambda i: (0, 0), memory_space=pltpu.SMEM),
        out_shape=jax.ShapeDtypeStruct((1, 2), jnp.float32),
    )(ctx3, tgt_rows, neg3)
    return partial[0, 0] / B + partial[0, 1] / (5 * B)


# speedup vs baseline: 9.7411x; 9.7411x over previous
import jax
import jax.numpy as jnp
from jax import lax
from jax.experimental import pallas as pl
from jax.experimental.pallas import tpu as pltpu
from jax.experimental.pallas import tpu_sc as plsc

EMB = 32
NC, NS = 2, 16
NW = NC * NS
TC_BLK = 2048


def _retile_body(in_ref, out_ref):
    w = in_ref[...]
    x = jnp.concatenate(
        [w[:, b * TC_BLK:(b + 1) * TC_BLK] for b in range(4)], axis=0)
    eye = (jax.lax.broadcasted_iota(jnp.int32, (128, 128), 0)
           == jax.lax.broadcasted_iota(jnp.int32, (128, 128), 1))
    out_ref[...] = lax.dot_general(
        x, eye.astype(jnp.float32), (((0,), (0,)), ((), ())),
        preferred_element_type=jnp.float32)


def _retile(W):
    V = W.shape[0]
    g = (V + 4 * TC_BLK - 1) // (4 * TC_BLK)
    Wt = W.T
    flat = pl.pallas_call(
        _retile_body,
        grid=(g,),
        in_specs=[pl.BlockSpec((EMB, 4 * TC_BLK), lambda i: (0, i))],
        out_specs=pl.BlockSpec((TC_BLK, 128), lambda i: (i, 0)),
        out_shape=jax.ShapeDtypeStruct((g * TC_BLK, 128), jnp.float32),
    )(Wt)
    return flat.reshape(g * TC_BLK * 4, EMB)


def _remap_idx(i):
    q = 4 * TC_BLK
    return (i & ~(q - 1)) | ((i & (TC_BLK - 1)) << 2) | ((i & (q - 1)) >> 11)


def _ctx_gather_body(ctx_hbm, win_hbm, ctx_out, slab_c, rows_v,
                     sem_i, sem_g, sem_o):
    B = ctx_hbm.shape[0] // 6
    n = B // NW
    wid = lax.axis_index("s") * NC + lax.axis_index("c")
    base = wid * n
    pltpu.async_copy(ctx_hbm.at[pl.ds(6 * base, 6 * n)], slab_c, sem_i).wait()
    out_copies = [None, None]
    for k in range(6):
        buf = k % 2
        if out_copies[buf] is not None:
            out_copies[buf].wait()
        idx = slab_c.at[pl.ds(k * n, n)]
        pltpu.async_copy(win_hbm.at[idx], rows_v.at[buf], sem_g).wait()
        out_copies[buf] = pltpu.async_copy(
            rows_v.at[buf], ctx_out.at[pl.ds(6 * base + k * n, n)], sem_o)
    out_copies[0].wait()
    out_copies[1].wait()


def _tn_gather_body(tgt_hbm, neg_hbm, wout_hbm, tgt_out, neg_out,
                    slab_t, slab_n, rows_v, sem_i, sem_g, sem_o):
    B = tgt_hbm.shape[0]
    n = B // NW
    wid = lax.axis_index("s") * NC + lax.axis_index("c")
    base = wid * n
    ti = pltpu.async_copy(tgt_hbm.at[pl.ds(base, n)], slab_t, sem_i)
    ni = pltpu.async_copy(neg_hbm.at[pl.ds(5 * base, 5 * n)], slab_n, sem_i)
    ti.wait()
    ni.wait()
    chunks = ([(slab_t, 0, tgt_out, base)]
              + [(slab_n, k, neg_out, 5 * base + k * n) for k in range(5)])
    out_copies = [None, None]
    for i, (slab, k, out, off) in enumerate(chunks):
        buf = i % 2
        if out_copies[buf] is not None:
            out_copies[buf].wait()
        idx = slab.at[pl.ds(k * n, n)]
        pltpu.async_copy(wout_hbm.at[idx], rows_v.at[buf], sem_g).wait()
        out_copies[buf] = pltpu.async_copy(
            rows_v.at[buf], out.at[pl.ds(off, n)], sem_o)
    out_copies[0].wait()
    out_copies[1].wait()


def _dense_body(ctx_ref, tgt_ref, neg_ref, out_ref):
    @pl.when(pl.program_id(0) == 0)
    def _():
        out_ref[0, 0] = jnp.float32(0.0)
        out_ref[0, 1] = jnp.float32(0.0)

    eps = 1e-12
    cm = jnp.sum(ctx_ref[...], axis=1) * (1.0 / 6.0)
    t = tgt_ref[...]
    tt = jnp.sum(t * t, axis=1)
    cc = jnp.sum(cm * cm, axis=1)
    tc = jnp.sum(t * cm, axis=1)
    rt = lax.rsqrt(jnp.maximum(tt, eps))
    rc = lax.rsqrt(jnp.maximum(cc, eps))
    pos = jnp.sum(jax.nn.sigmoid(tc * rt * rc))
    neg = jnp.float32(0.0)
    for j in range(5):
        nrow = neg_ref[:, j, :]
        nn = jnp.sum(nrow * nrow, axis=1)
        tn = jnp.sum(t * nrow, axis=1)
        rn = lax.rsqrt(jnp.maximum(nn, eps))
        neg = neg + jnp.sum(jax.nn.sigmoid(-(tn * rt * rn)))
    out_ref[0, 0] += pos
    out_ref[0, 1] += neg


def kernel(contexts, target, negatives, W_in, W_out):
    B = contexts.shape[0]
    n = B // NW
    mesh = plsc.VectorSubcoreMesh(core_axis_name="c", subcore_axis_name="s")

    win_rm = _retile(W_in)
    ctx_rows = pl.kernel(
        _ctx_gather_body,
        out_type=jax.ShapeDtypeStruct((6 * B, EMB), jnp.float32),
        mesh=mesh,
        scratch_types=[
            pltpu.VMEM((6 * n,), jnp.int32),
            pltpu.VMEM((2, n, EMB), jnp.float32),
            pltpu.SemaphoreType.DMA,
            pltpu.SemaphoreType.DMA,
            pltpu.SemaphoreType.DMA,
        ],
        compiler_params=pltpu.CompilerParams(use_tc_tiling_on_sc=False),
    )(_remap_idx(contexts.reshape(-1).astype(jnp.int32)), win_rm)

    wout_rm = _retile(W_out)
    tgt_rows, neg_rows = pl.kernel(
        _tn_gather_body,
        out_type=(
            jax.ShapeDtypeStruct((B, EMB), jnp.float32),
            jax.ShapeDtypeStruct((5 * B, EMB), jnp.float32),
        ),
        mesh=mesh,
        scratch_types=[
            pltpu.VMEM((n,), jnp.int32),
            pltpu.VMEM((5 * n,), jnp.int32),
            pltpu.VMEM((2, n, EMB), jnp.float32),
            pltpu.SemaphoreType.DMA,
            pltpu.SemaphoreType.DMA,
            pltpu.SemaphoreType.DMA,
        ],
        compiler_params=pltpu.CompilerParams(use_tc_tiling_on_sc=False),
    )(_remap_idx(target.reshape(-1).astype(jnp.int32)),
      _remap_idx(negatives.reshape(-1).astype(jnp.int32)), wout_rm)

    ctx3 = ctx_rows.reshape(B, 6, EMB)
    neg3 = neg_rows.reshape(B, 5, EMB)
    R = 2048
    partial = pl.pallas_call(
        _dense_body,
        grid=(B // R,),
        in_specs=[
            pl.BlockSpec((R, 6, EMB), lambda i: (i, 0, 0)),
            pl.BlockSpec((R, EMB), lambda i: (i, 0)),
            pl.BlockSpec((R, 5, EMB), lambda i: (i, 0, 0)),
        ],
        out_specs=pl.BlockSpec((1, 2), lambda i: (0, 0), memory_space=pltpu.SMEM),
        out_shape=jax.ShapeDtypeStruct((1, 2), jnp.float32),
    )(ctx3, tgt_rows, neg3)
    return partial[0, 0] / B + partial[0, 1] / (5 * B)


# speedup vs baseline: 15.7855x; 1.6205x over previous
import jax
import jax.numpy as jnp
from jax import lax
from jax.experimental import pallas as pl
from jax.experimental.pallas import tpu as pltpu
from jax.experimental.pallas import tpu_sc as plsc

EMB = 32
NC, NS = 2, 16
NW = NC * NS
TC_BLK = 2048


def _retile_body(in_ref, out_ref):
    w = in_ref[...]
    x = jnp.concatenate(
        [w[:, b * TC_BLK:(b + 1) * TC_BLK] for b in range(4)], axis=0)
    eye = (jax.lax.broadcasted_iota(jnp.int32, (128, 128), 0)
           == jax.lax.broadcasted_iota(jnp.int32, (128, 128), 1))
    out_ref[...] = lax.dot_general(
        x, eye.astype(jnp.float32), (((0,), (0,)), ((), ())),
        preferred_element_type=jnp.float32)


def _retile(W):
    V = W.shape[0]
    g = (V + 4 * TC_BLK - 1) // (4 * TC_BLK)
    Wt = W.T
    flat = pl.pallas_call(
        _retile_body,
        grid=(g,),
        in_specs=[pl.BlockSpec((EMB, 4 * TC_BLK), lambda i: (0, i))],
        out_specs=pl.BlockSpec((TC_BLK, 128), lambda i: (i, 0)),
        out_shape=jax.ShapeDtypeStruct((g * TC_BLK, 128), jnp.float32),
    )(Wt)
    return flat.reshape(g * TC_BLK * 4, EMB)


def _remap_idx(i):
    q = 4 * TC_BLK
    return (i & ~(q - 1)) | ((i & (TC_BLK - 1)) << 2) | ((i & (q - 1)) >> 11)


def _ctx_gather_body(ctx_hbm, win_hbm, ctx_out, slab_c, rows_v,
                     sem_i, sem_g, sem_o):
    B = ctx_hbm.shape[0] // 6
    n = B // NW
    wid = lax.axis_index("s") * NC + lax.axis_index("c")
    base = wid * n
    idx_copies = [
        pltpu.async_copy(ctx_hbm.at[pl.ds(j * B + base, n)],
                         slab_c.at[pl.ds(j * n, n)], sem_i)
        for j in range(6)
    ]
    out_copies = [None, None]
    for j in range(6):
        buf = j % 2
        if out_copies[buf] is not None:
            out_copies[buf].wait()
        idx_copies[j].wait()
        idx = slab_c.at[pl.ds(j * n, n)]
        pltpu.async_copy(win_hbm.at[idx], rows_v.at[buf], sem_g).wait()
        out_copies[buf] = pltpu.async_copy(
            rows_v.at[buf], ctx_out.at[pl.ds(j * B + base, n)], sem_o)
    out_copies[0].wait()
    out_copies[1].wait()


def _tn_gather_body(tgt_hbm, neg_hbm, wout_hbm, tgt_out, neg_out,
                    slab_t, slab_n, rows_v, sem_i, sem_g, sem_o):
    B = tgt_hbm.shape[0]
    n = B // NW
    wid = lax.axis_index("s") * NC + lax.axis_index("c")
    base = wid * n
    ti = pltpu.async_copy(tgt_hbm.at[pl.ds(base, n)], slab_t, sem_i)
    ni_copies = [
        pltpu.async_copy(neg_hbm.at[pl.ds(j * B + base, n)],
                         slab_n.at[pl.ds(j * n, n)], sem_i)
        for j in range(5)
    ]
    ti.wait()
    chunks = [(slab_t.at[pl.ds(0, n)], None, tgt_out, base)] + [
        (slab_n.at[pl.ds(j * n, n)], ni_copies[j], neg_out, j * B + base)
        for j in range(5)
    ]
    out_copies = [None, None]
    for i, (idx, ic, out, off) in enumerate(chunks):
        buf = i % 2
        if out_copies[buf] is not None:
            out_copies[buf].wait()
        if ic is not None:
            ic.wait()
        pltpu.async_copy(wout_hbm.at[idx], rows_v.at[buf], sem_g).wait()
        out_copies[buf] = pltpu.async_copy(
            rows_v.at[buf], out.at[pl.ds(off, n)], sem_o)
    out_copies[0].wait()
    out_copies[1].wait()


def _dense_body(c0, c1, c2, c3, c4, c5, t_ref, n0, n1, n2, n3, n4, out_ref):
    @pl.when(pl.program_id(0) == 0)
    def _():
        out_ref[0, 0] = jnp.float32(0.0)
        out_ref[0, 1] = jnp.float32(0.0)

    eps = 1e-12
    cm = (c0[...] + c1[...] + c2[...] + c3[...] + c4[...] + c5[...]) \
        * (1.0 / 6.0)
    t = t_ref[...]
    negs = [n0[...], n1[...], n2[...], n3[...], n4[...]]
    z = jnp.concatenate(
        [cm * cm, t * cm, t * t]
        + [t * nj for nj in negs] + [nj * nj for nj in negs], axis=0)
    sel = (jax.lax.broadcasted_iota(jnp.int32, (128, 4), 0) // 32
           == jax.lax.broadcasted_iota(jnp.int32, (128, 4), 1))
    d = lax.dot_general(z, sel.astype(jnp.float32), (((1,), (0,)), ((), ())),
                        preferred_element_type=jnp.float32)
    r4 = t.shape[0]
    cc, tc, tt = d[:r4], d[r4:2 * r4], d[2 * r4:3 * r4]
    tn, nn = d[3 * r4:8 * r4], d[8 * r4:13 * r4]
    rt = lax.rsqrt(jnp.maximum(tt, eps))
    cos_t = tc * rt * lax.rsqrt(jnp.maximum(cc, eps))
    rt5 = jnp.concatenate([rt] * 5, axis=0)
    cos_n = tn * rt5 * lax.rsqrt(jnp.maximum(nn, eps))
    out_ref[0, 0] += jnp.sum(jax.nn.sigmoid(cos_t))
    out_ref[0, 1] += jnp.sum(jax.nn.sigmoid(-cos_n))


def kernel(contexts, target, negatives, W_in, W_out):
    B = contexts.shape[0]
    n = B // NW
    mesh = plsc.VectorSubcoreMesh(core_axis_name="c", subcore_axis_name="s")

    win_rm = _retile(W_in)
    ctx_rows = pl.kernel(
        _ctx_gather_body,
        out_type=jax.ShapeDtypeStruct((6 * B, EMB), jnp.float32),
        mesh=mesh,
        scratch_types=[
            pltpu.VMEM((6 * n,), jnp.int32),
            pltpu.VMEM((2, n, EMB), jnp.float32),
            pltpu.SemaphoreType.DMA,
            pltpu.SemaphoreType.DMA,
            pltpu.SemaphoreType.DMA,
        ],
        compiler_params=pltpu.CompilerParams(use_tc_tiling_on_sc=False),
    )(_remap_idx(contexts.T.reshape(-1).astype(jnp.int32)), win_rm)

    wout_rm = _retile(W_out)
    tgt_rows, neg_rows = pl.kernel(
        _tn_gather_body,
        out_type=(
            jax.ShapeDtypeStruct((B, EMB), jnp.float32),
            jax.ShapeDtypeStruct((5 * B, EMB), jnp.float32),
        ),
        mesh=mesh,
        scratch_types=[
            pltpu.VMEM((n,), jnp.int32),
            pltpu.VMEM((5 * n,), jnp.int32),
            pltpu.VMEM((2, n, EMB), jnp.float32),
            pltpu.SemaphoreType.DMA,
            pltpu.SemaphoreType.DMA,
            pltpu.SemaphoreType.DMA,
        ],
        compiler_params=pltpu.CompilerParams(use_tc_tiling_on_sc=False),
    )(_remap_idx(target.reshape(-1).astype(jnp.int32)),
      _remap_idx(negatives.T.reshape(-1).astype(jnp.int32)), wout_rm)

    ctx_p = ctx_rows.reshape(6 * B // 4, 128)
    tgt_p = tgt_rows.reshape(B // 4, 128)
    neg_p = neg_rows.reshape(5 * B // 4, 128)

    R = 2048
    r4 = R // 4
    qb = (B // 4) // r4
    partial = pl.pallas_call(
        _dense_body,
        grid=(B // R,),
        in_specs=(
            [pl.BlockSpec((r4, 128), lambda i, j=j: (j * qb + i, 0))
             for j in range(6)]
            + [pl.BlockSpec((r4, 128), lambda i: (i, 0))]
            + [pl.BlockSpec((r4, 128), lambda i, j=j: (j * qb + i, 0))
               for j in range(5)]
        ),
        out_specs=pl.BlockSpec((1, 2), lambda i: (0, 0),
                               memory_space=pltpu.SMEM),
        out_shape=jax.ShapeDtypeStruct((1, 2), jnp.float32),
    )(ctx_p, ctx_p, ctx_p, ctx_p, ctx_p, ctx_p, tgt_p,
      neg_p, neg_p, neg_p, neg_p, neg_p)
    return partial[0, 0] / B + partial[0, 1] / (5 * B)


# speedup vs baseline: 22.9880x; 1.4563x over previous
import jax
import jax.numpy as jnp
from jax import lax
from jax.experimental import pallas as pl
from jax.experimental.pallas import tpu as pltpu
from jax.experimental.pallas import tpu_sc as plsc

EMB = 32
NC, NS = 2, 16
NW = NC * NS
TC_BLK = 8192


def _retile_body(in_ref, out_ref):
    w = in_ref[...]
    x = jnp.concatenate(
        [w[:, b * TC_BLK:(b + 1) * TC_BLK] for b in range(4)], axis=0)
    eye = (jax.lax.broadcasted_iota(jnp.int32, (128, 128), 0)
           == jax.lax.broadcasted_iota(jnp.int32, (128, 128), 1))
    out_ref[...] = lax.dot_general(
        x, eye.astype(jnp.float32), (((0,), (0,)), ((), ())),
        preferred_element_type=jnp.float32)


def _retile(W):
    V = W.shape[0]
    g = (V + 4 * TC_BLK - 1) // (4 * TC_BLK)
    Wt = W.T
    flat = pl.pallas_call(
        _retile_body,
        grid=(g,),
        in_specs=[pl.BlockSpec((EMB, 4 * TC_BLK), lambda i: (0, i))],
        out_specs=pl.BlockSpec((TC_BLK, 128), lambda i: (i, 0)),
        out_shape=jax.ShapeDtypeStruct((g * TC_BLK, 128), jnp.float32),
    )(Wt)
    return flat.reshape(g * TC_BLK * 4, EMB)


def _remap_idx(i):
    q = 4 * TC_BLK
    sh = TC_BLK.bit_length() - 1
    return (i & ~(q - 1)) | ((i & (TC_BLK - 1)) << 2) | ((i & (q - 1)) >> sh)


def _ctx_gather_body(ctx_hbm, win_hbm, ctx_out, slab_c, rows_v,
                     sem_i, sem_g, sem_o):
    B = ctx_hbm.shape[0] // 6
    n = B // NW
    wid = lax.axis_index("s") * NC + lax.axis_index("c")
    base = wid * n
    idx_copies = [
        pltpu.async_copy(ctx_hbm.at[pl.ds(j * B + base, n)],
                         slab_c.at[pl.ds(j * n, n)], sem_i)
        for j in range(6)
    ]
    out_copies = [None, None]
    for j in range(6):
        buf = j % 2
        if out_copies[buf] is not None:
            out_copies[buf].wait()
        idx_copies[j].wait()
        idx = slab_c.at[pl.ds(j * n, n)]
        pltpu.async_copy(win_hbm.at[idx], rows_v.at[buf], sem_g).wait()
        out_copies[buf] = pltpu.async_copy(
            rows_v.at[buf], ctx_out.at[pl.ds(j * B + base, n)], sem_o)
    out_copies[0].wait()
    out_copies[1].wait()


def _tn_gather_body(tgt_hbm, neg_hbm, wout_hbm, tgt_out, neg_out,
                    slab_t, slab_n, rows_v, sem_i, sem_g, sem_o):
    B = tgt_hbm.shape[0]
    n = B // NW
    wid = lax.axis_index("s") * NC + lax.axis_index("c")
    base = wid * n
    ti = pltpu.async_copy(tgt_hbm.at[pl.ds(base, n)], slab_t, sem_i)
    ni_copies = [
        pltpu.async_copy(neg_hbm.at[pl.ds(j * B + base, n)],
                         slab_n.at[pl.ds(j * n, n)], sem_i)
        for j in range(5)
    ]
    ti.wait()
    chunks = [(slab_t.at[pl.ds(0, n)], None, tgt_out, base)] + [
        (slab_n.at[pl.ds(j * n, n)], ni_copies[j], neg_out, j * B + base)
        for j in range(5)
    ]
    out_copies = [None, None]
    for i, (idx, ic, out, off) in enumerate(chunks):
        buf = i % 2
        if out_copies[buf] is not None:
            out_copies[buf].wait()
        if ic is not None:
            ic.wait()
        pltpu.async_copy(wout_hbm.at[idx], rows_v.at[buf], sem_g).wait()
        out_copies[buf] = pltpu.async_copy(
            rows_v.at[buf], out.at[pl.ds(off, n)], sem_o)
    out_copies[0].wait()
    out_copies[1].wait()


def _dense_body(c0, c1, c2, c3, c4, c5, t_ref, n0, n1, n2, n3, n4, out_ref):
    @pl.when(pl.program_id(0) == 0)
    def _():
        out_ref[0, 0] = jnp.float32(0.0)
        out_ref[0, 1] = jnp.float32(0.0)

    eps = 1e-12
    cm = (c0[...] + c1[...] + c2[...] + c3[...] + c4[...] + c5[...]) \
        * (1.0 / 6.0)
    t = t_ref[...]
    negs = [n0[...], n1[...], n2[...], n3[...], n4[...]]
    z = jnp.concatenate(
        [cm * cm, t * cm, t * t]
        + [t * nj for nj in negs] + [nj * nj for nj in negs], axis=0)
    sel = (jax.lax.broadcasted_iota(jnp.int32, (128, 4), 0) // 32
           == jax.lax.broadcasted_iota(jnp.int32, (128, 4), 1))
    d = lax.dot_general(z, sel.astype(jnp.float32), (((1,), (0,)), ((), ())),
                        preferred_element_type=jnp.float32)
    r4 = t.shape[0]
    cc, tc, tt = d[:r4], d[r4:2 * r4], d[2 * r4:3 * r4]
    tn, nn = d[3 * r4:8 * r4], d[8 * r4:13 * r4]
    rt = lax.rsqrt(jnp.maximum(tt, eps))
    cos_t = tc * rt * lax.rsqrt(jnp.maximum(cc, eps))
    rt5 = jnp.concatenate([rt] * 5, axis=0)
    cos_n = tn * rt5 * lax.rsqrt(jnp.maximum(nn, eps))
    out_ref[0, 0] += jnp.sum(jax.nn.sigmoid(cos_t))
    out_ref[0, 1] += jnp.sum(jax.nn.sigmoid(-cos_n))


def kernel(contexts, target, negatives, W_in, W_out):
    B = contexts.shape[0]
    n = B // NW
    mesh = plsc.VectorSubcoreMesh(core_axis_name="c", subcore_axis_name="s")

    win_rm = _retile(W_in)
    ctx_rows = pl.kernel(
        _ctx_gather_body,
        out_type=jax.ShapeDtypeStruct((6 * B, EMB), jnp.float32),
        mesh=mesh,
        scratch_types=[
            pltpu.VMEM((6 * n,), jnp.int32),
            pltpu.VMEM((2, n, EMB), jnp.float32),
            pltpu.SemaphoreType.DMA,
            pltpu.SemaphoreType.DMA,
            pltpu.SemaphoreType.DMA,
        ],
        compiler_params=pltpu.CompilerParams(use_tc_tiling_on_sc=False),
    )(_remap_idx(contexts.T.reshape(-1).astype(jnp.int32)), win_rm)

    wout_rm = _retile(W_out)
    tgt_rows, neg_rows = pl.kernel(
        _tn_gather_body,
        out_type=(
            jax.ShapeDtypeStruct((B, EMB), jnp.float32),
            jax.ShapeDtypeStruct((5 * B, EMB), jnp.float32),
        ),
        mesh=mesh,
        scratch_types=[
            pltpu.VMEM((n,), jnp.int32),
            pltpu.VMEM((5 * n,), jnp.int32),
            pltpu.VMEM((2, n, EMB), jnp.float32),
            pltpu.SemaphoreType.DMA,
            pltpu.SemaphoreType.DMA,
            pltpu.SemaphoreType.DMA,
        ],
        compiler_params=pltpu.CompilerParams(use_tc_tiling_on_sc=False),
    )(_remap_idx(target.reshape(-1).astype(jnp.int32)),
      _remap_idx(negatives.T.reshape(-1).astype(jnp.int32)), wout_rm)

    ctx_p = ctx_rows.reshape(6 * B // 4, 128)
    tgt_p = tgt_rows.reshape(B // 4, 128)
    neg_p = neg_rows.reshape(5 * B // 4, 128)

    R = 2048
    r4 = R // 4
    qb = (B // 4) // r4
    partial = pl.pallas_call(
        _dense_body,
        grid=(B // R,),
        in_specs=(
            [pl.BlockSpec((r4, 128), lambda i, j=j: (j * qb + i, 0))
             for j in range(6)]
            + [pl.BlockSpec((r4, 128), lambda i: (i, 0))]
            + [pl.BlockSpec((r4, 128), lambda i, j=j: (j * qb + i, 0))
               for j in range(5)]
        ),
        out_specs=pl.BlockSpec((1, 2), lambda i: (0, 0),
                               memory_space=pltpu.SMEM),
        out_shape=jax.ShapeDtypeStruct((1, 2), jnp.float32),
    )(ctx_p, ctx_p, ctx_p, ctx_p, ctx_p, ctx_p, tgt_p,
      neg_p, neg_p, neg_p, neg_p, neg_p)
    return partial[0, 0] / B + partial[0, 1] / (5 * B)


# speedup vs baseline: 23.0039x; 1.0007x over previous
import jax
import jax.numpy as jnp
from jax import lax
from jax.experimental import pallas as pl
from jax.experimental.pallas import tpu as pltpu
from jax.experimental.pallas import tpu_sc as plsc

EMB = 32
NC, NS = 2, 16
NW = NC * NS
TC_BLK = 16384


def _retile_body(in_ref, out_ref):
    w = in_ref[...]
    x = jnp.concatenate(
        [w[:, b * TC_BLK:(b + 1) * TC_BLK] for b in range(4)], axis=0)
    eye = (jax.lax.broadcasted_iota(jnp.int32, (128, 128), 0)
           == jax.lax.broadcasted_iota(jnp.int32, (128, 128), 1))
    out_ref[...] = lax.dot_general(
        x, eye.astype(jnp.float32), (((0,), (0,)), ((), ())),
        preferred_element_type=jnp.float32)


def _retile(W):
    V = W.shape[0]
    g = (V + 4 * TC_BLK - 1) // (4 * TC_BLK)
    Wt = W.T
    flat = pl.pallas_call(
        _retile_body,
        grid=(g,),
        in_specs=[pl.BlockSpec((EMB, 4 * TC_BLK), lambda i: (0, i))],
        out_specs=pl.BlockSpec((TC_BLK, 128), lambda i: (i, 0)),
        out_shape=jax.ShapeDtypeStruct((g * TC_BLK, 128), jnp.float32),
    )(Wt)
    return flat.reshape(g * TC_BLK * 4, EMB)


def _remap_idx(i):
    q = 4 * TC_BLK
    sh = TC_BLK.bit_length() - 1
    return (i & ~(q - 1)) | ((i & (TC_BLK - 1)) << 2) | ((i & (q - 1)) >> sh)


def _ctx_gather_body(ctx_hbm, win_hbm, ctx_out, slab_c, rows_v,
                     sem_i, sem_g, sem_o):
    B = ctx_hbm.shape[0] // 6
    n = B // NW
    wid = lax.axis_index("s") * NC + lax.axis_index("c")
    base = wid * n
    idx_copies = [
        pltpu.async_copy(ctx_hbm.at[pl.ds(j * B + base, n)],
                         slab_c.at[pl.ds(j * n, n)], sem_i)
        for j in range(6)
    ]
    out_copies = [None, None]
    for j in range(6):
        buf = j % 2
        if out_copies[buf] is not None:
            out_copies[buf].wait()
        idx_copies[j].wait()
        idx = slab_c.at[pl.ds(j * n, n)]
        pltpu.async_copy(win_hbm.at[idx], rows_v.at[buf], sem_g).wait()
        out_copies[buf] = pltpu.async_copy(
            rows_v.at[buf], ctx_out.at[pl.ds(j * B + base, n)], sem_o)
    out_copies[0].wait()
    out_copies[1].wait()


def _tn_gather_body(tgt_hbm, neg_hbm, wout_hbm, tgt_out, neg_out,
                    slab_t, slab_n, rows_v, sem_i, sem_g, sem_o):
    B = tgt_hbm.shape[0]
    n = B // NW
    wid = lax.axis_index("s") * NC + lax.axis_index("c")
    base = wid * n
    ti = pltpu.async_copy(tgt_hbm.at[pl.ds(base, n)], slab_t, sem_i)
    ni_copies = [
        pltpu.async_copy(neg_hbm.at[pl.ds(j * B + base, n)],
                         slab_n.at[pl.ds(j * n, n)], sem_i)
        for j in range(5)
    ]
    ti.wait()
    chunks = [(slab_t.at[pl.ds(0, n)], None, tgt_out, base)] + [
        (slab_n.at[pl.ds(j * n, n)], ni_copies[j], neg_out, j * B + base)
        for j in range(5)
    ]
    out_copies = [None, None]
    for i, (idx, ic, out, off) in enumerate(chunks):
        buf = i % 2
        if out_copies[buf] is not None:
            out_copies[buf].wait()
        if ic is not None:
            ic.wait()
        pltpu.async_copy(wout_hbm.at[idx], rows_v.at[buf], sem_g).wait()
        out_copies[buf] = pltpu.async_copy(
            rows_v.at[buf], out.at[pl.ds(off, n)], sem_o)
    out_copies[0].wait()
    out_copies[1].wait()


def _dense_body(c0, c1, c2, c3, c4, c5, t_ref, n0, n1, n2, n3, n4, out_ref):
    @pl.when(pl.program_id(0) == 0)
    def _():
        out_ref[0, 0] = jnp.float32(0.0)
        out_ref[0, 1] = jnp.float32(0.0)

    eps = 1e-12
    cm = (c0[...] + c1[...] + c2[...] + c3[...] + c4[...] + c5[...]) \
        * (1.0 / 6.0)
    t = t_ref[...]
    negs = [n0[...], n1[...], n2[...], n3[...], n4[...]]
    z = jnp.concatenate(
        [cm * cm, t * cm, t * t]
        + [t * nj for nj in negs] + [nj * nj for nj in negs], axis=0)
    sel = (jax.lax.broadcasted_iota(jnp.int32, (128, 4), 0) // 32
           == jax.lax.broadcasted_iota(jnp.int32, (128, 4), 1))
    d = lax.dot_general(z, sel.astype(jnp.float32), (((1,), (0,)), ((), ())),
                        preferred_element_type=jnp.float32)
    r4 = t.shape[0]
    cc, tc, tt = d[:r4], d[r4:2 * r4], d[2 * r4:3 * r4]
    tn, nn = d[3 * r4:8 * r4], d[8 * r4:13 * r4]
    rt = lax.rsqrt(jnp.maximum(tt, eps))
    cos_t = tc * rt * lax.rsqrt(jnp.maximum(cc, eps))
    rt5 = jnp.concatenate([rt] * 5, axis=0)
    cos_n = tn * rt5 * lax.rsqrt(jnp.maximum(nn, eps))
    out_ref[0, 0] += jnp.sum(jax.nn.sigmoid(cos_t))
    out_ref[0, 1] += jnp.sum(jax.nn.sigmoid(-cos_n))


def kernel(contexts, target, negatives, W_in, W_out):
    B = contexts.shape[0]
    n = B // NW
    mesh = plsc.VectorSubcoreMesh(core_axis_name="c", subcore_axis_name="s")

    win_rm = _retile(W_in)
    ctx_rows = pl.kernel(
        _ctx_gather_body,
        out_type=jax.ShapeDtypeStruct((6 * B, EMB), jnp.float32),
        mesh=mesh,
        scratch_types=[
            pltpu.VMEM((6 * n,), jnp.int32),
            pltpu.VMEM((2, n, EMB), jnp.float32),
            pltpu.SemaphoreType.DMA,
            pltpu.SemaphoreType.DMA,
            pltpu.SemaphoreType.DMA,
        ],
        compiler_params=pltpu.CompilerParams(use_tc_tiling_on_sc=False),
    )(_remap_idx(contexts.T.reshape(-1).astype(jnp.int32)), win_rm)

    wout_rm = _retile(W_out)
    tgt_rows, neg_rows = pl.kernel(
        _tn_gather_body,
        out_type=(
            jax.ShapeDtypeStruct((B, EMB), jnp.float32),
            jax.ShapeDtypeStruct((5 * B, EMB), jnp.float32),
        ),
        mesh=mesh,
        scratch_types=[
            pltpu.VMEM((n,), jnp.int32),
            pltpu.VMEM((5 * n,), jnp.int32),
            pltpu.VMEM((2, n, EMB), jnp.float32),
            pltpu.SemaphoreType.DMA,
            pltpu.SemaphoreType.DMA,
            pltpu.SemaphoreType.DMA,
        ],
        compiler_params=pltpu.CompilerParams(use_tc_tiling_on_sc=False),
    )(_remap_idx(target.reshape(-1).astype(jnp.int32)),
      _remap_idx(negatives.T.reshape(-1).astype(jnp.int32)), wout_rm)

    ctx_p = ctx_rows.reshape(6 * B // 4, 128)
    tgt_p = tgt_rows.reshape(B // 4, 128)
    neg_p = neg_rows.reshape(5 * B // 4, 128)

    R = 2048
    r4 = R // 4
    qb = (B // 4) // r4
    partial = pl.pallas_call(
        _dense_body,
        grid=(B // R,),
        in_specs=(
            [pl.BlockSpec((r4, 128), lambda i, j=j: (j * qb + i, 0))
             for j in range(6)]
            + [pl.BlockSpec((r4, 128), lambda i: (i, 0))]
            + [pl.BlockSpec((r4, 128), lambda i, j=j: (j * qb + i, 0))
               for j in range(5)]
        ),
        out_specs=pl.BlockSpec((1, 2), lambda i: (0, 0),
                               memory_space=pltpu.SMEM),
        out_shape=jax.ShapeDtypeStruct((1, 2), jnp.float32),
    )(ctx_p, ctx_p, ctx_p, ctx_p, ctx_p, ctx_p, tgt_p,
      neg_p, neg_p, neg_p, neg_p, neg_p)
    return partial[0, 0] / B + partial[0, 1] / (5 * B)


# speedup vs baseline: 23.2018x; 1.0086x over previous
import jax
import jax.numpy as jnp
from jax import lax
from jax.experimental import pallas as pl
from jax.experimental.pallas import tpu as pltpu
from jax.experimental.pallas import tpu_sc as plsc

EMB = 32
NC, NS = 2, 16
NW = NC * NS
TC_BLK = 16384


def _retile_body(in_ref, out_ref):
    w = in_ref[...]
    x = jnp.concatenate(
        [w[:, b * TC_BLK:(b + 1) * TC_BLK] for b in range(4)], axis=0)
    eye = (jax.lax.broadcasted_iota(jnp.int32, (128, 128), 0)
           == jax.lax.broadcasted_iota(jnp.int32, (128, 128), 1))
    out_ref[...] = lax.dot_general(
        x, eye.astype(jnp.float32), (((0,), (0,)), ((), ())),
        preferred_element_type=jnp.float32)


def _retile(W):
    V = W.shape[0]
    g = (V + 4 * TC_BLK - 1) // (4 * TC_BLK)
    Wt = W.T
    flat = pl.pallas_call(
        _retile_body,
        grid=(g,),
        in_specs=[pl.BlockSpec((EMB, 4 * TC_BLK), lambda i: (0, i))],
        out_specs=pl.BlockSpec((TC_BLK, 128), lambda i: (i, 0)),
        out_shape=jax.ShapeDtypeStruct((g * TC_BLK, 128), jnp.float32),
    )(Wt)
    return flat.reshape(g * TC_BLK * 4, EMB)


def _remap_idx(i):
    q = 4 * TC_BLK
    sh = TC_BLK.bit_length() - 1
    return (i & ~(q - 1)) | ((i & (TC_BLK - 1)) << 2) | ((i & (q - 1)) >> sh)


def _ctx_gather_body(ctx_hbm, win_hbm, ctx_out, slab_c, rows_v,
                     sem_i, sem_g, sem_o):
    B = ctx_hbm.shape[0] // 6
    n = B // NW
    wid = lax.axis_index("s") * NC + lax.axis_index("c")
    base = wid * n
    idx_copies = [
        pltpu.async_copy(ctx_hbm.at[pl.ds(j * B + base, n)],
                         slab_c.at[pl.ds(j * n, n)], sem_i)
        for j in range(6)
    ]
    out_copies = [None, None]
    for j in range(6):
        buf = j % 2
        if out_copies[buf] is not None:
            out_copies[buf].wait()
        idx_copies[j].wait()
        idx = slab_c.at[pl.ds(j * n, n)]
        pltpu.async_copy(win_hbm.at[idx], rows_v.at[buf], sem_g).wait()
        out_copies[buf] = pltpu.async_copy(
            rows_v.at[buf], ctx_out.at[pl.ds(j * B + base, n)], sem_o)
    out_copies[0].wait()
    out_copies[1].wait()


def _tn_gather_body(tgt_hbm, neg_hbm, wout_hbm, tgt_out, neg_out,
                    slab_t, slab_n, rows_v, sem_i, sem_g, sem_o):
    B = tgt_hbm.shape[0]
    n = B // NW
    wid = lax.axis_index("s") * NC + lax.axis_index("c")
    base = wid * n
    ti = pltpu.async_copy(tgt_hbm.at[pl.ds(base, n)], slab_t, sem_i)
    ni_copies = [
        pltpu.async_copy(neg_hbm.at[pl.ds(j * B + base, n)],
                         slab_n.at[pl.ds(j * n, n)], sem_i)
        for j in range(5)
    ]
    ti.wait()
    chunks = [(slab_t.at[pl.ds(0, n)], None, tgt_out, base)] + [
        (slab_n.at[pl.ds(j * n, n)], ni_copies[j], neg_out, j * B + base)
        for j in range(5)
    ]
    out_copies = [None, None]
    for i, (idx, ic, out, off) in enumerate(chunks):
        buf = i % 2
        if out_copies[buf] is not None:
            out_copies[buf].wait()
        if ic is not None:
            ic.wait()
        pltpu.async_copy(wout_hbm.at[idx], rows_v.at[buf], sem_g).wait()
        out_copies[buf] = pltpu.async_copy(
            rows_v.at[buf], out.at[pl.ds(off, n)], sem_o)
    out_copies[0].wait()
    out_copies[1].wait()


def _dense_body(c0, c1, c2, c3, c4, c5, t_ref, n0, n1, n2, n3, n4, out_ref):
    @pl.when(pl.program_id(0) == 0)
    def _():
        out_ref[0, 0] = jnp.float32(0.0)
        out_ref[0, 1] = jnp.float32(0.0)

    eps = 1e-12
    cm = (c0[...] + c1[...] + c2[...] + c3[...] + c4[...] + c5[...]) \
        * (1.0 / 6.0)
    t = t_ref[...]
    negs = [n0[...], n1[...], n2[...], n3[...], n4[...]]
    z = jnp.concatenate(
        [cm * cm, t * cm, t * t]
        + [t * nj for nj in negs] + [nj * nj for nj in negs], axis=0)
    sel = (jax.lax.broadcasted_iota(jnp.int32, (128, 4), 0) // 32
           == jax.lax.broadcasted_iota(jnp.int32, (128, 4), 1))
    d = lax.dot_general(z, sel.astype(jnp.float32), (((1,), (0,)), ((), ())),
                        preferred_element_type=jnp.float32)
    r4 = t.shape[0]
    cc, tc, tt = d[:r4], d[r4:2 * r4], d[2 * r4:3 * r4]
    tn, nn = d[3 * r4:8 * r4], d[8 * r4:13 * r4]
    rt = lax.rsqrt(jnp.maximum(tt, eps))
    cos_t = tc * rt * lax.rsqrt(jnp.maximum(cc, eps))
    rt5 = jnp.concatenate([rt] * 5, axis=0)
    cos_n = tn * rt5 * lax.rsqrt(jnp.maximum(nn, eps))
    out_ref[0, 0] += jnp.sum(jax.nn.sigmoid(cos_t))
    out_ref[0, 1] += jnp.sum(jax.nn.sigmoid(-cos_n))


def kernel(contexts, target, negatives, W_in, W_out):
    B = contexts.shape[0]
    n = B // NW
    mesh = plsc.VectorSubcoreMesh(core_axis_name="c", subcore_axis_name="s")

    win_rm = _retile(W_in)
    ctx_rows = pl.kernel(
        _ctx_gather_body,
        out_type=jax.ShapeDtypeStruct((6 * B, EMB), jnp.float32),
        mesh=mesh,
        scratch_types=[
            pltpu.VMEM((6 * n,), jnp.int32),
            pltpu.VMEM((2, n, EMB), jnp.float32),
            pltpu.SemaphoreType.DMA,
            pltpu.SemaphoreType.DMA,
            pltpu.SemaphoreType.DMA,
        ],
        compiler_params=pltpu.CompilerParams(use_tc_tiling_on_sc=False),
    )(_remap_idx(contexts.T.reshape(-1).astype(jnp.int32)), win_rm)

    wout_rm = _retile(W_out)
    tgt_rows, neg_rows = pl.kernel(
        _tn_gather_body,
        out_type=(
            jax.ShapeDtypeStruct((B, EMB), jnp.float32),
            jax.ShapeDtypeStruct((5 * B, EMB), jnp.float32),
        ),
        mesh=mesh,
        scratch_types=[
            pltpu.VMEM((n,), jnp.int32),
            pltpu.VMEM((5 * n,), jnp.int32),
            pltpu.VMEM((2, n, EMB), jnp.float32),
            pltpu.SemaphoreType.DMA,
            pltpu.SemaphoreType.DMA,
            pltpu.SemaphoreType.DMA,
        ],
        compiler_params=pltpu.CompilerParams(use_tc_tiling_on_sc=False),
    )(_remap_idx(target.reshape(-1).astype(jnp.int32)),
      _remap_idx(negatives.T.reshape(-1).astype(jnp.int32)), wout_rm)

    ctx_p = ctx_rows.reshape(6 * B // 4, 128)
    tgt_p = tgt_rows.reshape(B // 4, 128)
    neg_p = neg_rows.reshape(5 * B // 4, 128)

    R = 4096
    r4 = R // 4
    qb = (B // 4) // r4
    partial = pl.pallas_call(
        _dense_body,
        grid=(B // R,),
        in_specs=(
            [pl.BlockSpec((r4, 128), lambda i, j=j: (j * qb + i, 0))
             for j in range(6)]
            + [pl.BlockSpec((r4, 128), lambda i: (i, 0))]
            + [pl.BlockSpec((r4, 128), lambda i, j=j: (j * qb + i, 0))
               for j in range(5)]
        ),
        out_specs=pl.BlockSpec((1, 2), lambda i: (0, 0),
                               memory_space=pltpu.SMEM),
        out_shape=jax.ShapeDtypeStruct((1, 2), jnp.float32),
    )(ctx_p, ctx_p, ctx_p, ctx_p, ctx_p, ctx_p, tgt_p,
      neg_p, neg_p, neg_p, neg_p, neg_p)
    return partial[0, 0] / B + partial[0, 1] / (5 * B)
